# baseline (speedup 1.0000x reference)
import jax
import jax.numpy as jnp
from jax.experimental import pallas as pl
from jax.experimental.pallas import tpu as pltpu

E = 8
TOPK = 2
D = 1024
DFF = 2048
T = 2048

BT = 128
NA = T * TOPK
NB = NA // BT + E
NP = NB * BT
EMAXB = T // BT
DH = DFF // 2
BC = 128


def _routing_kernel(g_ref, pos_ref, w_ref, nblk_ref, base_ref):
    s = g_ref[...]
    lane = jax.lax.broadcasted_iota(jnp.int32, (T, E), 1)
    m = jnp.max(s, axis=1, keepdims=True)
    p = jnp.exp(s - m)
    m1 = jnp.max(p, axis=1, keepdims=True)
    is1 = jnp.min(jnp.where(p == m1, lane, E), axis=1, keepdims=True)
    pm = jnp.where(lane == is1, -1.0, p)
    m2 = jnp.max(pm, axis=1, keepdims=True)
    is2 = jnp.min(jnp.where(pm == m2, lane, E), axis=1, keepdims=True)
    wsum = m1 + m2
    w_ref[:, 0:1] = m1 / wsum
    w_ref[:, 1:2] = m2 / wsum

    onehot = ((lane == is1) | (lane == is2)).astype(jnp.int32)
    c = onehot
    sft = 1
    while sft < T:
        z = jnp.zeros((sft, E), jnp.int32)
        c = c + jnp.concatenate([z, c[:T - sft, :]], axis=0)
        sft *= 2
    counts = c[T - 1:T, :]
    rank = c - onehot
    blocks_e = (counts + BT - 1) // BT
    b = blocks_e
    sft = 1
    while sft < E:
        z = jnp.zeros((1, sft), jnp.int32)
        b = b + jnp.concatenate([z, b[:, :E - sft]], axis=1)
        sft *= 2
    base_excl = b - blocks_e
    nblk_ref[...] = blocks_e
    base_ref[...] = base_excl

    def pick(isel):
        r = jnp.sum(jnp.where(lane == isel, rank, 0), axis=1, keepdims=True)
        bb = jnp.sum(jnp.where(lane == isel, base_excl, 0), axis=1,
                     keepdims=True)
        return bb * BT + r
    pos_ref[:, 0:1] = pick(is1)
    pos_ref[:, 1:2] = pick(is2)


def _expert_kernel(nblk_ref, base_ref, pos_ref, x_ref, w1g_ref, w1u_ref,
                   w2_ref, y_ref, xs_ref):
    e = pl.program_id(0)
    f = pl.program_id(1)
    nb = nblk_ref[e]
    base = base_ref[e]

    @pl.when((e == 0) & (f == 0))
    def _():
        y_ref[...] = jnp.zeros_like(y_ref)

    wg = w1g_ref[0]
    wu = w1u_ref[0]
    w2c = w2_ref[0]

    @pl.when(f == 0)
    def _():
        def gbody(i, _):
            row0 = (base + i) * BT
            row_id = jax.lax.broadcasted_iota(jnp.int32, (BT, T), 0) + row0
            sel = ((row_id == pos_ref[0, 0, :][None, :]) |
                   (row_id == pos_ref[0, 1, :][None, :])
                   ).astype(jnp.float32).astype(jnp.bfloat16)
            xs_ref[i] = jax.lax.dot_general(
                sel, x_ref[...], (((1,), (0,)), ((), ())),
                preferred_element_type=jnp.float32).astype(jnp.bfloat16)
            return 0
        jax.lax.fori_loop(0, nb, gbody, 0)

    nt = (((1,), (1,)), ((), ()))

    def cbody(i, _):
        xb = xs_ref[i]
        g = jax.lax.dot_general(xb, wg, nt, preferred_element_type=jnp.float32)
        u = jax.lax.dot_general(xb, wu, nt, preferred_element_type=jnp.float32)
        h = ((g * jax.nn.sigmoid(g)) * u).astype(jnp.bfloat16)
        yp = jax.lax.dot_general(h, w2c, nt, preferred_element_type=jnp.float32)
        blk = base + i
        prev = y_ref[blk].astype(jnp.float32)
        y_ref[blk] = jnp.where(f == 0, yp, prev + yp).astype(jnp.bfloat16)
        return 0
    jax.lax.fori_loop(0, nb, cbody, 0)


def _combine_kernel(pos_ref, w_ref, y_ref, out_ref):
    col_id = jax.lax.broadcasted_iota(jnp.int32, (BC, NP), 1)
    sel = (jnp.where(col_id == pos_ref[:, 0:1], w_ref[:, 0:1], 0.0) +
           jnp.where(col_id == pos_ref[:, 1:2], w_ref[:, 1:2], 0.0)
           ).astype(jnp.bfloat16)
    out_ref[...] = jax.lax.dot_general(sel, y_ref[...],
                                       (((1,), (0,)), ((), ())),
                                       preferred_element_type=jnp.float32)


@jax.jit
def kernel(x, gating_output, w1, w2):
    pos_tok, topk_w, nblk, base = pl.pallas_call(
        _routing_kernel,
        grid=(1,),
        in_specs=[pl.BlockSpec((T, E), lambda i: (0, 0))],
        out_specs=[
            pl.BlockSpec((T, TOPK), lambda i: (0, 0)),
            pl.BlockSpec((T, TOPK), lambda i: (0, 0)),
            pl.BlockSpec((1, E), lambda i: (0, 0)),
            pl.BlockSpec((1, E), lambda i: (0, 0)),
        ],
        out_shape=[
            jax.ShapeDtypeStruct((T, TOPK), jnp.int32),
            jax.ShapeDtypeStruct((T, TOPK), jnp.float32),
            jax.ShapeDtypeStruct((1, E), jnp.int32),
            jax.ShapeDtypeStruct((1, E), jnp.int32),
        ],
    )(gating_output.astype(jnp.float32))

    pos_t = pos_tok.T.reshape(1, TOPK, T)
    xb16 = x.astype(jnp.bfloat16)

    grid_spec = pltpu.PrefetchScalarGridSpec(
        num_scalar_prefetch=2,
        grid=(E, 2),
        in_specs=[
            pl.BlockSpec((1, TOPK, T), lambda e, f, nb, bs: (0, 0, 0)),
            pl.BlockSpec((T, D), lambda e, f, nb, bs: (0, 0)),
            pl.BlockSpec((1, DH, D), lambda e, f, nb, bs: (e, f, 0)),
            pl.BlockSpec((1, DH, D), lambda e, f, nb, bs: (e, 2 + f, 0)),
            pl.BlockSpec((1, D, DH), lambda e, f, nb, bs: (e, 0, f)),
        ],
        out_specs=pl.BlockSpec((NB, BT, D), lambda e, f, nb, bs: (0, 0, 0)),
        scratch_shapes=[pltpu.VMEM((EMAXB, BT, D), jnp.bfloat16)],
    )
    y_sorted = pl.pallas_call(
        _expert_kernel,
        grid_spec=grid_spec,
        out_shape=jax.ShapeDtypeStruct((NB, BT, D), jnp.bfloat16),
        compiler_params=pltpu.CompilerParams(
            dimension_semantics=("arbitrary", "arbitrary"),
        ),
    )(nblk.reshape(E), base.reshape(E), pos_t, xb16,
      w1.astype(jnp.bfloat16), w1.astype(jnp.bfloat16),
      w2.astype(jnp.bfloat16))

    return y_sorted.reshape(NP, D)[:T].astype(jnp.float32) + topk_w.sum()
    out = pl.pallas_call(
        _combine_kernel,
        grid=(T // BC,),
        in_specs=[
            pl.BlockSpec((BC, TOPK), lambda c: (c, 0)),
            pl.BlockSpec((BC, TOPK), lambda c: (c, 0)),
            pl.BlockSpec((NP, D), lambda c: (0, 0)),
        ],
        out_specs=pl.BlockSpec((BC, D), lambda c: (c, 0)),
        out_shape=jax.ShapeDtypeStruct((T, D), jnp.float32),
        compiler_params=pltpu.CompilerParams(
            dimension_semantics=("arbitrary",),
        ),
    )(pos_tok, topk_w, y_sorted.reshape(NP, D))
    return out


# speedup vs baseline: 1.1546x; 1.1546x over previous
import jax
import jax.numpy as jnp
from jax.experimental import pallas as pl
from jax.experimental.pallas import tpu as pltpu

E = 8
TOPK = 2
D = 1024
DFF = 2048
T = 2048

BT = 128
NA = T * TOPK
NB = NA // BT + E
NP = NB * BT
EMAXB = T // BT
NF = 4
DH = DFF // NF
BC = 128


def _routing_kernel(g_ref, pos_ref, w_ref, nblk_ref, base_ref):
    s = g_ref[...]
    lane = jax.lax.broadcasted_iota(jnp.int32, (T, E), 1)
    m = jnp.max(s, axis=1, keepdims=True)
    p = jnp.exp(s - m)
    m1 = jnp.max(p, axis=1, keepdims=True)
    is1 = jnp.min(jnp.where(p == m1, lane, E), axis=1, keepdims=True)
    pm = jnp.where(lane == is1, -1.0, p)
    m2 = jnp.max(pm, axis=1, keepdims=True)
    is2 = jnp.min(jnp.where(pm == m2, lane, E), axis=1, keepdims=True)
    wsum = m1 + m2
    w_ref[:, 0:1] = m1 / wsum
    w_ref[:, 1:2] = m2 / wsum

    onehot = ((lane == is1) | (lane == is2)).astype(jnp.int32)
    c = onehot
    sft = 1
    while sft < T:
        z = jnp.zeros((sft, E), jnp.int32)
        c = c + jnp.concatenate([z, c[:T - sft, :]], axis=0)
        sft *= 2
    counts = c[T - 1:T, :]
    rank = c - onehot
    blocks_e = (counts + BT - 1) // BT
    b = blocks_e
    sft = 1
    while sft < E:
        z = jnp.zeros((1, sft), jnp.int32)
        b = b + jnp.concatenate([z, b[:, :E - sft]], axis=1)
        sft *= 2
    base_excl = b - blocks_e
    nblk_ref[...] = blocks_e
    base_ref[...] = base_excl

    def pick(isel):
        r = jnp.sum(jnp.where(lane == isel, rank, 0), axis=1, keepdims=True)
        bb = jnp.sum(jnp.where(lane == isel, base_excl, 0), axis=1,
                     keepdims=True)
        return bb * BT + r
    pos_ref[:, 0:1] = pick(is1)
    pos_ref[:, 1:2] = pick(is2)


def _expert_kernel(nblk_ref, base_ref, pos_ref, x_ref, w1g_ref, w1u_ref,
                   w2_ref, y_ref, xs_ref):
    e = pl.program_id(0)
    f = pl.program_id(1)
    nb = nblk_ref[e]
    base = base_ref[e]

    @pl.when((e == 0) & (f == 0))
    def _():
        y_ref[...] = jnp.zeros_like(y_ref)

    wg = w1g_ref[0].astype(jnp.bfloat16)
    wu = w1u_ref[0].astype(jnp.bfloat16)
    w2c = w2_ref[0].astype(jnp.bfloat16)

    @pl.when(f == 0)
    def _():
        def gbody(i, _):
            row0 = (base + i) * BT
            row_id = jax.lax.broadcasted_iota(jnp.int32, (BT, T), 0) + row0
            sel = ((row_id == pos_ref[0, 0, :][None, :]) |
                   (row_id == pos_ref[0, 1, :][None, :])
                   ).astype(jnp.float32).astype(jnp.bfloat16)
            xs_ref[i] = jax.lax.dot_general(
                sel, x_ref[...], (((1,), (0,)), ((), ())),
                preferred_element_type=jnp.float32).astype(jnp.bfloat16)
            return 0
        jax.lax.fori_loop(0, nb, gbody, 0)

    nt = (((1,), (1,)), ((), ()))

    def cbody(i, _):
        xb = xs_ref[i]
        g = jax.lax.dot_general(xb, wg, nt, preferred_element_type=jnp.float32)
        u = jax.lax.dot_general(xb, wu, nt, preferred_element_type=jnp.float32)
        h = ((g * jax.nn.sigmoid(g)) * u).astype(jnp.bfloat16)
        yp = jax.lax.dot_general(h, w2c, nt, preferred_element_type=jnp.float32)
        blk = base + i
        prev = y_ref[blk].astype(jnp.float32)
        y_ref[blk] = jnp.where(f == 0, yp, prev + yp).astype(jnp.bfloat16)
        return 0
    jax.lax.fori_loop(0, nb, cbody, 0)


def _combine_kernel(pos_ref, w_ref, y_ref, out_ref):
    col_id = jax.lax.broadcasted_iota(jnp.int32, (BC, NP), 1)
    sel = (jnp.where(col_id == pos_ref[:, 0:1], w_ref[:, 0:1], 0.0) +
           jnp.where(col_id == pos_ref[:, 1:2], w_ref[:, 1:2], 0.0)
           ).astype(jnp.bfloat16)
    out_ref[...] = jax.lax.dot_general(sel, y_ref[...],
                                       (((1,), (0,)), ((), ())),
                                       preferred_element_type=jnp.float32)


@jax.jit
def kernel(x, gating_output, w1, w2):
    pos_tok, topk_w, nblk, base = pl.pallas_call(
        _routing_kernel,
        grid=(1,),
        in_specs=[pl.BlockSpec((T, E), lambda i: (0, 0))],
        out_specs=[
            pl.BlockSpec((T, TOPK), lambda i: (0, 0)),
            pl.BlockSpec((T, TOPK), lambda i: (0, 0)),
            pl.BlockSpec((1, E), lambda i: (0, 0)),
            pl.BlockSpec((1, E), lambda i: (0, 0)),
        ],
        out_shape=[
            jax.ShapeDtypeStruct((T, TOPK), jnp.int32),
            jax.ShapeDtypeStruct((T, TOPK), jnp.float32),
            jax.ShapeDtypeStruct((1, E), jnp.int32),
            jax.ShapeDtypeStruct((1, E), jnp.int32),
        ],
    )(gating_output.astype(jnp.float32))

    pos_t = pos_tok.T.reshape(1, TOPK, T)
    xb16 = x.astype(jnp.bfloat16)

    grid_spec = pltpu.PrefetchScalarGridSpec(
        num_scalar_prefetch=2,
        grid=(E, NF),
        in_specs=[
            pl.BlockSpec((1, TOPK, T), lambda e, f, nb, bs: (0, 0, 0)),
            pl.BlockSpec((T, D), lambda e, f, nb, bs: (0, 0)),
            pl.BlockSpec((1, DH, D), lambda e, f, nb, bs: (e, f, 0)),
            pl.BlockSpec((1, DH, D), lambda e, f, nb, bs: (e, NF + f, 0)),
            pl.BlockSpec((1, D, DH), lambda e, f, nb, bs: (e, 0, f)),
        ],
        out_specs=pl.BlockSpec((NB, BT, D), lambda e, f, nb, bs: (0, 0, 0)),
        scratch_shapes=[pltpu.VMEM((EMAXB, BT, D), jnp.bfloat16)],
    )
    y_sorted = pl.pallas_call(
        _expert_kernel,
        grid_spec=grid_spec,
        out_shape=jax.ShapeDtypeStruct((NB, BT, D), jnp.bfloat16),
        compiler_params=pltpu.CompilerParams(
            dimension_semantics=("arbitrary", "arbitrary"),
        ),
    )(nblk.reshape(E), base.reshape(E), pos_t, xb16, w1, w1, w2)

    out = pl.pallas_call(
        _combine_kernel,
        grid=(T // BC,),
        in_specs=[
            pl.BlockSpec((BC, TOPK), lambda c: (c, 0)),
            pl.BlockSpec((BC, TOPK), lambda c: (c, 0)),
            pl.BlockSpec((NP, D), lambda c: (0, 0)),
        ],
        out_specs=pl.BlockSpec((BC, D), lambda c: (c, 0)),
        out_shape=jax.ShapeDtypeStruct((T, D), jnp.float32),
        compiler_params=pltpu.CompilerParams(
            dimension_semantics=("arbitrary",),
        ),
    )(pos_tok, topk_w, y_sorted.reshape(NP, D))
    return out


# speedup vs baseline: 1.2578x; 1.0894x over previous
import jax
import jax.numpy as jnp
from jax.experimental import pallas as pl
from jax.experimental.pallas import tpu as pltpu

E = 8
TOPK = 2
D = 1024
DFF = 2048
T = 2048

BT = 128
NA = T * TOPK
NB = NA // BT + E
NP = NB * BT
EMAXB = T // BT
NF = 2
DH = DFF // NF
BC = 128
NS = E * NF


def _routing_kernel(g_ref, pos_ref, w_ref, nblk_ref, base_ref):
    s = g_ref[...]
    lane = jax.lax.broadcasted_iota(jnp.int32, (T, E), 1)
    m = jnp.max(s, axis=1, keepdims=True)
    p = jnp.exp(s - m)
    m1 = jnp.max(p, axis=1, keepdims=True)
    is1 = jnp.min(jnp.where(p == m1, lane, E), axis=1, keepdims=True)
    pm = jnp.where(lane == is1, -1.0, p)
    m2 = jnp.max(pm, axis=1, keepdims=True)
    is2 = jnp.min(jnp.where(pm == m2, lane, E), axis=1, keepdims=True)
    wsum = m1 + m2
    w_ref[:, 0:1] = m1 / wsum
    w_ref[:, 1:2] = m2 / wsum

    onehot = ((lane == is1) | (lane == is2)).astype(jnp.int32)
    c = onehot
    sft = 1
    while sft < T:
        z = jnp.zeros((sft, E), jnp.int32)
        c = c + jnp.concatenate([z, c[:T - sft, :]], axis=0)
        sft *= 2
    counts = c[T - 1:T, :]
    rank = c - onehot
    blocks_e = (counts + BT - 1) // BT
    b = blocks_e
    sft = 1
    while sft < E:
        z = jnp.zeros((1, sft), jnp.int32)
        b = b + jnp.concatenate([z, b[:, :E - sft]], axis=1)
        sft *= 2
    base_excl = b - blocks_e
    nblk_ref[...] = blocks_e
    base_ref[...] = base_excl

    def pick(isel):
        r = jnp.sum(jnp.where(lane == isel, rank, 0), axis=1, keepdims=True)
        bb = jnp.sum(jnp.where(lane == isel, base_excl, 0), axis=1,
                     keepdims=True)
        return bb * BT + r
    pos_ref[:, 0:1] = pick(is1)
    pos_ref[:, 1:2] = pick(is2)


def _wcopies(w1_hbm, w2_hbm, wg_buf, wu_buf, w2_buf, sem, s, buf):
    e = s // NF
    f = s % NF
    cg = pltpu.make_async_copy(
        w1_hbm.at[e, pl.ds(f * DH, DH), :], wg_buf.at[buf], sem.at[buf, 0])
    cu = pltpu.make_async_copy(
        w1_hbm.at[e, pl.ds(DFF + f * DH, DH), :], wu_buf.at[buf],
        sem.at[buf, 1])
    c2 = pltpu.make_async_copy(
        w2_hbm.at[e, :, pl.ds(f * DH, DH)], w2_buf.at[buf], sem.at[buf, 2])
    return cg, cu, c2


def _expert_kernel(nblk_ref, base_ref, pos_ref, x_ref, w1_hbm, w2_hbm,
                   y_ref, xs_ref, wg_buf, wu_buf, w2_buf, sem):
    e = pl.program_id(0)
    f = pl.program_id(1)
    s = e * NF + f
    nb = nblk_ref[e]
    base = base_ref[e]

    @pl.when(s == 0)
    def _():
        y_ref[...] = jnp.zeros_like(y_ref)
        for c in _wcopies(w1_hbm, w2_hbm, wg_buf, wu_buf, w2_buf, sem, 0, 0):
            c.start()

    @pl.when(s + 1 < NS)
    def _():
        for c in _wcopies(w1_hbm, w2_hbm, wg_buf, wu_buf, w2_buf, sem,
                          s + 1, (s + 1) % 2):
            c.start()

    @pl.when(f == 0)
    def _():
        def gbody(i, _):
            row0 = (base + i) * BT
            row_id = jax.lax.broadcasted_iota(jnp.int32, (BT, T), 0) + row0
            sel = ((row_id == pos_ref[0, 0, :][None, :]) |
                   (row_id == pos_ref[0, 1, :][None, :])
                   ).astype(jnp.float32).astype(jnp.bfloat16)
            xs_ref[i] = jax.lax.dot_general(
                sel, x_ref[...], (((1,), (0,)), ((), ())),
                preferred_element_type=jnp.float32).astype(jnp.bfloat16)
            return 0
        jax.lax.fori_loop(0, nb, gbody, 0)

    for c in _wcopies(w1_hbm, w2_hbm, wg_buf, wu_buf, w2_buf, sem, s, s % 2):
        c.wait()
    wg = wg_buf[s % 2].astype(jnp.bfloat16)
    wu = wu_buf[s % 2].astype(jnp.bfloat16)
    w2c = w2_buf[s % 2].astype(jnp.bfloat16)

    nt = (((1,), (1,)), ((), ()))

    def cbody(i, _):
        xb = xs_ref[i]
        g = jax.lax.dot_general(xb, wg, nt, preferred_element_type=jnp.float32)
        u = jax.lax.dot_general(xb, wu, nt, preferred_element_type=jnp.float32)
        h = ((g * jax.nn.sigmoid(g)) * u).astype(jnp.bfloat16)
        yp = jax.lax.dot_general(h, w2c, nt, preferred_element_type=jnp.float32)
        blk = base + i
        prev = y_ref[blk].astype(jnp.float32)
        y_ref[blk] = jnp.where(f == 0, yp, prev + yp).astype(jnp.bfloat16)
        return 0
    jax.lax.fori_loop(0, nb, cbody, 0)


def _combine_kernel(pos_ref, w_ref, y_ref, out_ref):
    col_id = jax.lax.broadcasted_iota(jnp.int32, (BC, NP), 1)
    sel = (jnp.where(col_id == pos_ref[:, 0:1], w_ref[:, 0:1], 0.0) +
           jnp.where(col_id == pos_ref[:, 1:2], w_ref[:, 1:2], 0.0)
           ).astype(jnp.bfloat16)
    out_ref[...] = jax.lax.dot_general(sel, y_ref[...],
                                       (((1,), (0,)), ((), ())),
                                       preferred_element_type=jnp.float32)


@jax.jit
def kernel(x, gating_output, w1, w2):
    pos_tok, topk_w, nblk, base = pl.pallas_call(
        _routing_kernel,
        grid=(1,),
        in_specs=[pl.BlockSpec((T, E), lambda i: (0, 0))],
        out_specs=[
            pl.BlockSpec((T, TOPK), lambda i: (0, 0)),
            pl.BlockSpec((T, TOPK), lambda i: (0, 0)),
            pl.BlockSpec((1, E), lambda i: (0, 0)),
            pl.BlockSpec((1, E), lambda i: (0, 0)),
        ],
        out_shape=[
            jax.ShapeDtypeStruct((T, TOPK), jnp.int32),
            jax.ShapeDtypeStruct((T, TOPK), jnp.float32),
            jax.ShapeDtypeStruct((1, E), jnp.int32),
            jax.ShapeDtypeStruct((1, E), jnp.int32),
        ],
    )(gating_output.astype(jnp.float32))

    pos_t = pos_tok.T.reshape(1, TOPK, T)
    xb16 = x.astype(jnp.bfloat16)

    grid_spec = pltpu.PrefetchScalarGridSpec(
        num_scalar_prefetch=2,
        grid=(E, NF),
        in_specs=[
            pl.BlockSpec((1, TOPK, T), lambda e, f, nb, bs: (0, 0, 0)),
            pl.BlockSpec((T, D), lambda e, f, nb, bs: (0, 0)),
            pl.BlockSpec(memory_space=pl.ANY),
            pl.BlockSpec(memory_space=pl.ANY),
        ],
        out_specs=pl.BlockSpec((NB, BT, D), lambda e, f, nb, bs: (0, 0, 0)),
        scratch_shapes=[
            pltpu.VMEM((EMAXB, BT, D), jnp.bfloat16),
            pltpu.VMEM((2, DH, D), jnp.float32),
            pltpu.VMEM((2, DH, D), jnp.float32),
            pltpu.VMEM((2, D, DH), jnp.float32),
            pltpu.SemaphoreType.DMA((2, 3)),
        ],
    )
    y_sorted = pl.pallas_call(
        _expert_kernel,
        grid_spec=grid_spec,
        out_shape=jax.ShapeDtypeStruct((NB, BT, D), jnp.bfloat16),
        compiler_params=pltpu.CompilerParams(
            dimension_semantics=("arbitrary", "arbitrary"),
        ),
    )(nblk.reshape(E), base.reshape(E), pos_t, xb16, w1, w2)

    out = pl.pallas_call(
        _combine_kernel,
        grid=(T // BC,),
        in_specs=[
            pl.BlockSpec((BC, TOPK), lambda c: (c, 0)),
            pl.BlockSpec((BC, TOPK), lambda c: (c, 0)),
            pl.BlockSpec((NP, D), lambda c: (0, 0)),
        ],
        out_specs=pl.BlockSpec((BC, D), lambda c: (c, 0)),
        out_shape=jax.ShapeDtypeStruct((T, D), jnp.float32),
        compiler_params=pltpu.CompilerParams(
            dimension_semantics=("arbitrary",),
        ),
    )(pos_tok, topk_w, y_sorted.reshape(NP, D))
    return out


# speedup vs baseline: 1.7371x; 1.3810x over previous
import jax
import jax.numpy as jnp
from jax.experimental import pallas as pl
from jax.experimental.pallas import tpu as pltpu

E = 8
TOPK = 2
D = 1024
DFF = 2048
T = 2048

BT = 128
NA = T * TOPK
NB = NA // BT + E
NP = NB * BT
EMAXB = T // BT
NF = 2
DH = DFF // NF
BC = 128
NS = E * NF
NG = 4
NBA = NB + NG - 1


def _routing_kernel(g_ref, pos_ref, w_ref, nblk_ref, base_ref):
    s = g_ref[...]
    lane = jax.lax.broadcasted_iota(jnp.int32, (T, E), 1)
    m = jnp.max(s, axis=1, keepdims=True)
    p = jnp.exp(s - m)
    m1 = jnp.max(p, axis=1, keepdims=True)
    is1 = jnp.min(jnp.where(p == m1, lane, E), axis=1, keepdims=True)
    pm = jnp.where(lane == is1, -1.0, p)
    m2 = jnp.max(pm, axis=1, keepdims=True)
    is2 = jnp.min(jnp.where(pm == m2, lane, E), axis=1, keepdims=True)
    wsum = m1 + m2
    w_ref[:, 0:1] = m1 / wsum
    w_ref[:, 1:2] = m2 / wsum

    onehot = ((lane == is1) | (lane == is2)).astype(jnp.int32)
    c = onehot
    sft = 1
    while sft < T:
        z = jnp.zeros((sft, E), jnp.int32)
        c = c + jnp.concatenate([z, c[:T - sft, :]], axis=0)
        sft *= 2
    counts = c[T - 1:T, :]
    rank = c - onehot
    blocks_e = (counts + BT - 1) // BT
    b = blocks_e
    sft = 1
    while sft < E:
        z = jnp.zeros((1, sft), jnp.int32)
        b = b + jnp.concatenate([z, b[:, :E - sft]], axis=1)
        sft *= 2
    base_excl = b - blocks_e
    nblk_ref[...] = blocks_e
    base_ref[...] = base_excl

    def pick(isel):
        r = jnp.sum(jnp.where(lane == isel, rank, 0), axis=1, keepdims=True)
        bb = jnp.sum(jnp.where(lane == isel, base_excl, 0), axis=1,
                     keepdims=True)
        return bb * BT + r
    pos_ref[:, 0:1] = pick(is1)
    pos_ref[:, 1:2] = pick(is2)


def _wcopies(w1_hbm, w2_hbm, wg_buf, wu_buf, w2_buf, sem, s, buf):
    e = s // NF
    f = s % NF
    cg = pltpu.make_async_copy(
        w1_hbm.at[e, pl.ds(f * DH, DH), :], wg_buf.at[buf], sem.at[buf, 0])
    cu = pltpu.make_async_copy(
        w1_hbm.at[e, pl.ds(DFF + f * DH, DH), :], wu_buf.at[buf],
        sem.at[buf, 1])
    c2 = pltpu.make_async_copy(
        w2_hbm.at[e, :, pl.ds(f * DH, DH)], w2_buf.at[buf], sem.at[buf, 2])
    return cg, cu, c2


def _expert_kernel(nblk_ref, base_ref, pos_ref, x_ref, w1_hbm, w2_hbm,
                   y_ref, xs_ref, wg_buf, wu_buf, w2_buf, sem):
    e = pl.program_id(0)
    f = pl.program_id(1)
    s = e * NF + f
    nb = nblk_ref[e]
    base = base_ref[e]

    @pl.when(s == 0)
    def _():
        y_ref[...] = jnp.zeros_like(y_ref)
        for c in _wcopies(w1_hbm, w2_hbm, wg_buf, wu_buf, w2_buf, sem, 0, 0):
            c.start()

    @pl.when(s + 1 < NS)
    def _():
        for c in _wcopies(w1_hbm, w2_hbm, wg_buf, wu_buf, w2_buf, sem,
                          s + 1, (s + 1) % 2):
            c.start()

    @pl.when(f == 0)
    def _():
        def gbody(i, _):
            row0 = (base + i) * BT
            row_id = jax.lax.broadcasted_iota(jnp.int32, (BT, T), 0) + row0
            sel = ((row_id == pos_ref[0, 0, :][None, :]) |
                   (row_id == pos_ref[0, 1, :][None, :])
                   ).astype(jnp.float32).astype(jnp.bfloat16)
            xs_ref[i] = jax.lax.dot_general(
                sel, x_ref[...], (((1,), (0,)), ((), ())),
                preferred_element_type=jnp.float32).astype(jnp.bfloat16)
            return 0
        jax.lax.fori_loop(0, nb, gbody, 0)

        def zbody(i, _):
            xs_ref[i] = jnp.zeros((BT, D), jnp.bfloat16)
            return 0
        jax.lax.fori_loop(nb, ((nb + NG - 1) // NG) * NG, zbody, 0)

    for c in _wcopies(w1_hbm, w2_hbm, wg_buf, wu_buf, w2_buf, sem, s, s % 2):
        c.wait()
    wg = wg_buf[s % 2].astype(jnp.bfloat16)
    wu = wu_buf[s % 2].astype(jnp.bfloat16)
    w2c = w2_buf[s % 2].astype(jnp.bfloat16)

    nt = (((1,), (1,)), ((), ()))

    def cbody(gi, _):
        xb = xs_ref[pl.ds(gi * NG, NG)].reshape(NG * BT, D)
        g = jax.lax.dot_general(xb, wg, nt, preferred_element_type=jnp.float32)
        u = jax.lax.dot_general(xb, wu, nt, preferred_element_type=jnp.float32)
        h = ((g * jax.nn.sigmoid(g)) * u).astype(jnp.bfloat16)
        yp = jax.lax.dot_general(h, w2c, nt, preferred_element_type=jnp.float32)
        yp4 = yp.reshape(NG, BT, D)
        blk = base + gi * NG
        prev = y_ref[pl.ds(blk, NG)].astype(jnp.float32)
        y_ref[pl.ds(blk, NG)] = jnp.where(f == 0, yp4,
                                          prev + yp4).astype(jnp.bfloat16)
        return 0
    jax.lax.fori_loop(0, (nb + NG - 1) // NG, cbody, 0)


def _combine_kernel(pos_ref, w_ref, y_ref, out_ref):
    col_id = jax.lax.broadcasted_iota(jnp.int32, (BC, NP), 1)
    sel = (jnp.where(col_id == pos_ref[:, 0:1], w_ref[:, 0:1], 0.0) +
           jnp.where(col_id == pos_ref[:, 1:2], w_ref[:, 1:2], 0.0)
           ).astype(jnp.bfloat16)
    out_ref[...] = jax.lax.dot_general(sel, y_ref[...],
                                       (((1,), (0,)), ((), ())),
                                       preferred_element_type=jnp.float32)


@jax.jit
def kernel(x, gating_output, w1, w2):
    pos_tok, topk_w, nblk, base = pl.pallas_call(
        _routing_kernel,
        grid=(1,),
        in_specs=[pl.BlockSpec((T, E), lambda i: (0, 0))],
        out_specs=[
            pl.BlockSpec((T, TOPK), lambda i: (0, 0)),
            pl.BlockSpec((T, TOPK), lambda i: (0, 0)),
            pl.BlockSpec((1, E), lambda i: (0, 0)),
            pl.BlockSpec((1, E), lambda i: (0, 0)),
        ],
        out_shape=[
            jax.ShapeDtypeStruct((T, TOPK), jnp.int32),
            jax.ShapeDtypeStruct((T, TOPK), jnp.float32),
            jax.ShapeDtypeStruct((1, E), jnp.int32),
            jax.ShapeDtypeStruct((1, E), jnp.int32),
        ],
    )(gating_output.astype(jnp.float32))

    pos_t = pos_tok.T.reshape(1, TOPK, T)
    xb16 = x.astype(jnp.bfloat16)

    grid_spec = pltpu.PrefetchScalarGridSpec(
        num_scalar_prefetch=2,
        grid=(E, NF),
        in_specs=[
            pl.BlockSpec((1, TOPK, T), lambda e, f, nb, bs: (0, 0, 0)),
            pl.BlockSpec((T, D), lambda e, f, nb, bs: (0, 0)),
            pl.BlockSpec(memory_space=pl.ANY),
            pl.BlockSpec(memory_space=pl.ANY),
        ],
        out_specs=pl.BlockSpec((NBA, BT, D), lambda e, f, nb, bs: (0, 0, 0)),
        scratch_shapes=[
            pltpu.VMEM((EMAXB, BT, D), jnp.bfloat16),
            pltpu.VMEM((2, DH, D), jnp.float32),
            pltpu.VMEM((2, DH, D), jnp.float32),
            pltpu.VMEM((2, D, DH), jnp.float32),
            pltpu.SemaphoreType.DMA((2, 3)),
        ],
    )
    y_sorted = pl.pallas_call(
        _expert_kernel,
        grid_spec=grid_spec,
        out_shape=jax.ShapeDtypeStruct((NBA, BT, D), jnp.bfloat16),
        compiler_params=pltpu.CompilerParams(
            dimension_semantics=("arbitrary", "arbitrary"),
        ),
    )(nblk.reshape(E), base.reshape(E), pos_t, xb16, w1, w2)

    out = pl.pallas_call(
        _combine_kernel,
        grid=(T // BC,),
        in_specs=[
            pl.BlockSpec((BC, TOPK), lambda c: (c, 0)),
            pl.BlockSpec((BC, TOPK), lambda c: (c, 0)),
            pl.BlockSpec((NP, D), lambda c: (0, 0)),
        ],
        out_specs=pl.BlockSpec((BC, D), lambda c: (c, 0)),
        out_shape=jax.ShapeDtypeStruct((T, D), jnp.float32),
        compiler_params=pltpu.CompilerParams(
            dimension_semantics=("arbitrary",),
        ),
    )(pos_tok, topk_w, y_sorted[:NB].reshape(NP, D))
    return out
